# baseline (speedup 1.0000x reference)
import functools

import jax
import jax.numpy as jnp
from jax import lax
from jax.experimental import pallas as pl
from jax.experimental.pallas import tpu as pltpu
from jax.experimental.pallas import tpu_sc as plsc

NC = 2
NS = 16
NW = NC * NS

IDXW = 128
DPAD = 128
SUPER_IR = 8
SUB = IDXW
NSLOT = 4
RETIRE_LAG = 2
PERIOD = 2 * SUPER_IR


def _make_gather(n_rows_total, d):
    rows_per_w = n_rows_total // NW
    ir_per_w = rows_per_w // IDXW
    n_super = ir_per_w // SUPER_IR
    mesh = plsc.VectorSubcoreMesh(core_axis_name="c", subcore_axis_name="s")

    @functools.partial(
        pl.kernel,
        mesh=mesh,
        out_type=jax.ShapeDtypeStruct((n_rows_total, d), jnp.float32),
        scratch_types=[
            pltpu.VMEM((SUPER_IR, IDXW), jnp.int32),
            pltpu.VMEM((SUPER_IR, IDXW), jnp.int32),
            pltpu.VMEM((SUB, DPAD), jnp.float32),
            pltpu.VMEM((SUB, DPAD), jnp.float32),
            pltpu.VMEM((SUB, DPAD), jnp.float32),
            pltpu.VMEM((SUB, DPAD), jnp.float32),
            pltpu.SemaphoreType.DMA,
            pltpu.SemaphoreType.DMA,
            pltpu.SemaphoreType.DMA,
            pltpu.SemaphoreType.DMA,
            pltpu.SemaphoreType.DMA,
            pltpu.SemaphoreType.DMA,
            pltpu.SemaphoreType.DMA,
            pltpu.SemaphoreType.DMA,
        ],
        compiler_params=pltpu.CompilerParams(use_tc_tiling_on_sc=False),
    )
    def gather_kernel(table_hbm, idx_hbm, out_hbm, idx_v0, idx_v1,
                      r0, r1, r2, r3, sg0, sg1, sg2, sg3,
                      so0, so1, so2, so3):
        wid = lax.axis_index("s") * NC + lax.axis_index("c")
        base_ir = wid * ir_per_w
        base_row = wid * rows_per_w
        idx_v = (idx_v0, idx_v1)
        rows = (r0, r1, r2, r3)
        sg = (sg0, sg1, sg2, sg3)
        so = (so0, so1, so2, so3)

        def idx_load(sc, ibuf):
            pltpu.sync_copy(
                idx_hbm.at[pl.ds(base_ir + sc * SUPER_IR, SUPER_IR)],
                idx_v[ibuf])

        def gather_desc(slot, ibuf, sub):
            return (table_hbm.at[idx_v[ibuf].at[sub]], rows[slot], sg[slot])

        def wb_desc(t, slot):
            return (rows[slot].at[:, pl.ds(0, d)],
                    out_hbm.at[pl.ds(base_row + t * SUB, SUB)], so[slot])

        def step(t, k, do_free, do_retire):
            slot = k % NSLOT
            ibuf = (k // SUPER_IR) % 2
            if do_free:
                a, b, s = wb_desc(t - NSLOT, slot)
                pltpu.make_async_copy(a, b, s).wait()
            if k % SUPER_IR == 0:
                idx_load(t // SUPER_IR, ibuf)
            a, b, s = gather_desc(slot, ibuf, k % SUPER_IR)
            pltpu.async_copy(a, b, s)
            if do_retire:
                k2 = (k - RETIRE_LAG) % PERIOD
                a, b, s = gather_desc(k2 % NSLOT, (k2 // SUPER_IR) % 2,
                                      k2 % SUPER_IR)
                pltpu.make_async_copy(a, b, s).wait()
                a, b, s = wb_desc(t - RETIRE_LAG, k2 % NSLOT)
                pltpu.async_copy(a, b, s)

        for k in range(PERIOD):
            step(k, k, do_free=(k >= NSLOT), do_retire=(k >= RETIRE_LAG))

        def body(p, carry):
            t0 = PERIOD + p * PERIOD
            for k in range(PERIOD):
                step(t0 + k, k, do_free=True, do_retire=True)
            return carry

        lax.fori_loop(0, (n_super - 3) // 2, body, 0)

        t0 = (n_super - 1) * SUPER_IR
        for k in range(SUPER_IR):
            step(t0 + k, k, do_free=True, do_retire=True)
        for k in range(SUPER_IR, SUPER_IR + RETIRE_LAG):
            k2 = (k - RETIRE_LAG) % PERIOD
            a, b, s = gather_desc(k2 % NSLOT, 0, k2 % SUPER_IR)
            pltpu.make_async_copy(a, b, s).wait()
            a, b, s = wb_desc(t0 + k - RETIRE_LAG, k2 % NSLOT)
            pltpu.async_copy(a, b, s)
        for k in range(SUPER_IR - NSLOT, SUPER_IR):
            a, b, s = wb_desc(t0 + k, k % NSLOT)
            pltpu.make_async_copy(a, b, s).wait()

    return gather_kernel


def kernel(x, table):
    b, h = x.shape
    v, d = table.shape
    n = b * h
    idx2d = x.reshape(n // IDXW, IDXW).astype(jnp.int32)
    tpad = jnp.pad(table, ((0, 0), (0, DPAD - d)))
    out = _make_gather(n, d)(tpad, idx2d)
    return out.reshape(b, h, d)
